# baseline (speedup 1.0000x reference)
import numpy as np
import jax
import jax.numpy as jnp
from jax import lax
from jax.experimental import pallas as pl
from jax.experimental.pallas import tpu as pltpu
from jax.experimental.pallas import tpu_sc as plsc

_L = 16
_NW = 32
_PT = 128
_CCH = 256

_SORT8 = [(0, 1), (2, 3), (4, 5), (6, 7),
          (0, 2), (1, 3), (4, 6), (5, 7),
          (1, 2), (5, 6), (0, 4), (3, 7),
          (1, 5), (2, 6),
          (1, 4), (3, 6),
          (2, 4), (3, 5),
          (3, 4)]

_BITONIC8 = [(0, 4), (1, 5), (2, 6), (3, 7),
             (0, 2), (1, 3), (4, 6), (5, 7),
             (0, 1), (2, 3), (4, 5), (6, 7)]


def _sort8(v):
    v = list(v)
    for a, b in _SORT8:
        hi = jnp.maximum(v[a], v[b])
        lo = jnp.minimum(v[a], v[b])
        v[a] = hi
        v[b] = lo
    return v


def _merge8(a, b):
    m = [jnp.maximum(a[j], b[7 - j]) for j in range(8)]
    for p, q in _BITONIC8:
        hi = jnp.maximum(m[p], m[q])
        lo = jnp.minimum(m[p], m[q])
        m[p] = hi
        m[q] = lo
    return m


def _make_sc_call(B, CD, HW):
    NFT = HW // _PT
    NTASK = B * NFT
    NCH = CD // _CCH
    ntw = NTASK // _NW
    assert NTASK % _NW == 0 and CD % _CCH == 0

    def body(x_hbm, tk_hbm, mn_hbm, buf, accs, stg_tk, stg_mn):
        nc = plsc.get_sparse_core_info().num_cores
        wid = lax.axis_index("s") * nc + lax.axis_index("c")

        def task_body(k, carry):
            tid = wid + k * _NW
            b = tid // NFT
            p0 = pl.multiple_of((tid % NFT) * _PT, _PT)

            for c in range(NCH):
                pltpu.sync_copy(
                    x_hbm.at[b, pl.ds(c * _CCH, _CCH), pl.ds(p0, _PT)],
                    buf)

                def lg_body(lg, carry2):
                    o = lg * _L
                    if c == 0:
                        ninf = jnp.full((_L,), -jnp.inf, jnp.float32)
                        acc = [ninf] * 8
                        tot = jnp.zeros((_L,), jnp.float32)
                    else:
                        acc = [accs[lg, j, :] for j in range(8)]
                        tot = accs[lg, 8, :]

                    def blk_body(i, cr):
                        acc = list(cr[:8])
                        tot = cr[8]
                        r0 = 32 * i
                        v = [buf[r0 + j, pl.ds(o, _L)] for j in range(32)]
                        s = v[0]
                        for j in range(1, 32):
                            s = s + v[j]
                        tot = tot + s
                        t1 = _merge8(_sort8(v[0:8]), _sort8(v[8:16]))
                        t2 = _merge8(_sort8(v[16:24]), _sort8(v[24:32]))
                        acc = _merge8(acc, _merge8(t1, t2))
                        return (*acc, tot)

                    out = lax.fori_loop(0, _CCH // 32, blk_body,
                                        (*acc, tot))
                    for j in range(8):
                        accs[lg, j, :] = out[j]
                    accs[lg, 8, :] = out[8]
                    return carry2

                lax.fori_loop(0, _PT // _L, lg_body, 0)

            def fin_body(lg, carry2):
                o = lg * _L
                a = [accs[lg, j, :] for j in range(8)]
                tk = ((a[0] + a[1]) + (a[2] + a[3])) + \
                     ((a[4] + a[5]) + (a[6] + a[7]))
                stg_tk[0, pl.ds(o, _L)] = tk
                stg_mn[0, pl.ds(o, _L)] = accs[lg, 8, :] * (1.0 / CD)
                return carry2

            lax.fori_loop(0, _PT // _L, fin_body, 0)
            pltpu.sync_copy(stg_tk, tk_hbm.at[wid, k])
            pltpu.sync_copy(stg_mn, mn_hbm.at[wid, k])
            return carry

        lax.fori_loop(0, ntw, task_body, 0)

    mesh = plsc.VectorSubcoreMesh(core_axis_name="c", subcore_axis_name="s")
    return pl.kernel(
        body,
        out_type=[jax.ShapeDtypeStruct((_NW, ntw, 8, _PT), jnp.float32),
                  jax.ShapeDtypeStruct((_NW, ntw, 8, _PT), jnp.float32)],
        mesh=mesh,
        scratch_types=[pltpu.VMEM((_CCH, _PT), jnp.float32),
                       pltpu.VMEM((8, 9, _L), jnp.float32),
                       pltpu.VMEM((8, _PT), jnp.float32),
                       pltpu.VMEM((8, _PT), jnp.float32)],
    ), NFT, ntw


def _tail_tc_kernel(x_ref, tk_ref, mn_ref):
    v = x_ref[0]
    cd = v.shape[0]
    mn_ref[0, 0, :] = jnp.sum(v, axis=0) * (1.0 / cd)
    acc = jnp.zeros((v.shape[1],), jnp.float32)
    rem = jnp.full((v.shape[1],), 8.0, jnp.float32)
    for _ in range(8):
        m = jnp.max(v, axis=0)
        cnt = jnp.sum((v == m[None, :]).astype(jnp.float32), axis=0)
        take = jnp.minimum(cnt, rem)
        acc = acc + jnp.where(take > 0, m * take, 0.0)
        rem = rem - take
        v = jnp.where(v == m[None, :], -jnp.inf, v)
    tk_ref[0, 0, :] = acc


def _make_tail_call(B, CD, HW, NFT):
    TW = HW - NFT * _PT
    grid = (B,)
    return pl.pallas_call(
        _tail_tc_kernel,
        grid=grid,
        in_specs=[pl.BlockSpec((1, CD, _PT), lambda i: (i, 0, NFT))],
        out_specs=[pl.BlockSpec((1, 1, _PT), lambda i: (i, 0, 0)),
                   pl.BlockSpec((1, 1, _PT), lambda i: (i, 0, 0))],
        out_shape=[jax.ShapeDtypeStruct((B, 1, _PT), jnp.float32),
                   jax.ShapeDtypeStruct((B, 1, _PT), jnp.float32)],
    ), TW


def _pixel_perm(B, NFT, ntw):
    bidx, pidx = np.meshgrid(np.arange(B), np.arange(NFT * _PT),
                             indexing="ij")
    tid = bidx * NFT + pidx // _PT
    w = tid % _NW
    tl = tid // _NW
    return (w * ntw + tl) * _PT + pidx % _PT


def kernel(x):
    b, c, d, h, w = x.shape
    cd, hw = c * d, h * w
    x3 = x.reshape(b, cd, hw)
    sc_call, NFT, ntw = _make_sc_call(b, cd, hw)
    tail_call, TW = _make_tail_call(b, cd, hw, NFT)
    tk, mn = sc_call(x3)
    ttk, tmn = tail_call(x3)
    perm = jnp.asarray(_pixel_perm(b, NFT, ntw))
    tk = jnp.take(tk[:, :, 0, :].reshape(-1), perm, axis=0)
    mn = jnp.take(mn[:, :, 0, :].reshape(-1), perm, axis=0)
    tk = jnp.concatenate([tk, ttk[:, 0, :TW]], axis=1)
    mn = jnp.concatenate([mn, tmn[:, 0, :TW]], axis=1)
    return (tk.reshape(b, 1, 1, h, w), mn.reshape(b, 1, 1, h, w))


# speedup vs baseline: 1.3619x; 1.3619x over previous
import jax
import jax.numpy as jnp
from jax import lax
from jax.experimental import pallas as pl
from jax.experimental.pallas import tpu as pltpu
from jax.experimental.pallas import tpu_sc as plsc

_L = 16
_NW = 32
_HT = 8
_CCH = 32

_SORT8 = [(0, 1), (2, 3), (4, 5), (6, 7),
          (0, 2), (1, 3), (4, 6), (5, 7),
          (1, 2), (5, 6), (0, 4), (3, 7),
          (1, 5), (2, 6),
          (1, 4), (3, 6),
          (2, 4), (3, 5),
          (3, 4)]

_BITONIC8 = [(0, 4), (1, 5), (2, 6), (3, 7),
             (0, 2), (1, 3), (4, 6), (5, 7),
             (0, 1), (2, 3), (4, 5), (6, 7)]


def _sort8(v):
    v = list(v)
    for a, b in _SORT8:
        hi = jnp.maximum(v[a], v[b])
        lo = jnp.minimum(v[a], v[b])
        v[a] = hi
        v[b] = lo
    return v


def _merge8(a, b):
    m = [jnp.maximum(a[j], b[7 - j]) for j in range(8)]
    for p, q in _BITONIC8:
        hi = jnp.maximum(m[p], m[q])
        lo = jnp.minimum(m[p], m[q])
        m[p] = hi
        m[q] = lo
    return m


def _make_sc_call(B, CD, H, W):
    NHT = H // _HT
    NTASK = B * NHT
    NCH = CD // _CCH
    NLGR = (W + _L - 1) // _L
    NLG = _HT * NLGR
    assert NTASK <= _NW and H % _HT == 0 and CD % _CCH == 0

    def body(x_hbm, tk_hbm, mn_hbm, buf, accs, stg_tk, stg_mn):
        nc = plsc.get_sparse_core_info().num_cores
        wid = lax.axis_index("s") * nc + lax.axis_index("c")

        @pl.when(wid < NTASK)
        def _():
            b = wid // NHT
            h0 = pl.multiple_of((wid % NHT) * _HT, _HT)

            def init_body(lg, carry2):
                ninf = jnp.full((_L,), -jnp.inf, jnp.float32)
                for j2 in range(8):
                    accs[lg, j2, :] = ninf
                accs[lg, 8, :] = jnp.zeros((_L,), jnp.float32)
                return carry2

            lax.fori_loop(0, NLG, init_body, 0)

            def chunk_body(c, carry):
                pltpu.sync_copy(
                    x_hbm.at[b, pl.ds(c * _CCH, _CCH), pl.ds(h0, _HT), :],
                    buf)

                def lg_body(lg, carry2):
                    hh = lg // NLGR
                    j = lg % NLGR
                    o = jnp.minimum(j * _L, W - _L)
                    acc = [accs[lg, j2, :] for j2 in range(8)]
                    tot = accs[lg, 8, :]

                    def blk_body(i, cr):
                        acc = list(cr[:8])
                        tot = cr[8]
                        r0 = 32 * i
                        v = [buf[r0 + j3, hh, pl.ds(o, _L)]
                             for j3 in range(32)]
                        s = v[0]
                        for j3 in range(1, 32):
                            s = s + v[j3]
                        tot = tot + s
                        t1 = _merge8(_sort8(v[0:8]), _sort8(v[8:16]))
                        t2 = _merge8(_sort8(v[16:24]), _sort8(v[24:32]))
                        acc = _merge8(acc, _merge8(t1, t2))
                        return (*acc, tot)

                    out = lax.fori_loop(0, _CCH // 32, blk_body,
                                        (*acc, tot))
                    for j2 in range(8):
                        accs[lg, j2, :] = out[j2]
                    accs[lg, 8, :] = out[8]
                    return carry2

                lax.fori_loop(0, NLG, lg_body, 0)
                return carry

            lax.fori_loop(0, NCH, chunk_body, 0)

            def fin_body(lg, carry2):
                hh = lg // NLGR
                j = lg % NLGR
                o = jnp.minimum(j * _L, W - _L)
                a = [accs[lg, j2, :] for j2 in range(8)]
                tk = ((a[0] + a[1]) + (a[2] + a[3])) + \
                     ((a[4] + a[5]) + (a[6] + a[7]))
                stg_tk[hh, pl.ds(o, _L)] = tk
                stg_mn[hh, pl.ds(o, _L)] = accs[lg, 8, :] * (1.0 / CD)
                return carry2

            lax.fori_loop(0, NLG, fin_body, 0)
            pltpu.sync_copy(stg_tk, tk_hbm.at[b, wid % NHT])
            pltpu.sync_copy(stg_mn, mn_hbm.at[b, wid % NHT])

    mesh = plsc.VectorSubcoreMesh(core_axis_name="c", subcore_axis_name="s")
    return pl.kernel(
        body,
        out_type=[jax.ShapeDtypeStruct((B, NHT, _HT, W), jnp.float32),
                  jax.ShapeDtypeStruct((B, NHT, _HT, W), jnp.float32)],
        mesh=mesh,
        scratch_types=[pltpu.VMEM((_CCH, _HT, W), jnp.float32),
                       pltpu.VMEM((NLG, 9, _L), jnp.float32),
                       pltpu.VMEM((_HT, W), jnp.float32),
                       pltpu.VMEM((_HT, W), jnp.float32)],
    )


def kernel(x):
    b, c, d, h, w = x.shape
    x4 = x.reshape(b, c * d, h, w)
    tk, mn = _make_sc_call(b, c * d, h, w)(x4)
    return (tk.reshape(b, 1, 1, h, w), mn.reshape(b, 1, 1, h, w))


# speedup vs baseline: 2.7272x; 2.0025x over previous
import jax
import jax.numpy as jnp
from jax import lax
from jax.experimental import pallas as pl
from jax.experimental.pallas import tpu as pltpu
from jax.experimental.pallas import tpu_sc as plsc

_L = 16
_NW = 32
_WCH = 8

_SORT8 = [(0, 1), (2, 3), (4, 5), (6, 7),
          (0, 2), (1, 3), (4, 6), (5, 7),
          (1, 2), (5, 6), (0, 4), (3, 7),
          (1, 5), (2, 6),
          (1, 4), (3, 6),
          (2, 4), (3, 5),
          (3, 4)]

_BITONIC8 = [(0, 4), (1, 5), (2, 6), (3, 7),
             (0, 2), (1, 3), (4, 6), (5, 7),
             (0, 1), (2, 3), (4, 5), (6, 7)]


def _sort8(v):
    v = list(v)
    for a, b in _SORT8:
        hi = jnp.maximum(v[a], v[b])
        lo = jnp.minimum(v[a], v[b])
        v[a] = hi
        v[b] = lo
    return v


def _merge8(a, b):
    m = [jnp.maximum(a[j], b[7 - j]) for j in range(8)]
    for p, q in _BITONIC8:
        hi = jnp.maximum(m[p], m[q])
        lo = jnp.minimum(m[p], m[q])
        m[p] = hi
        m[q] = lo
    return m


def _xmerge(a, b):
    return jnp.sort(jnp.maximum(a, jnp.flip(b)))


def _make_sc_call(B, C, D, H, W):
    NT = B * H
    NTW = NT // _NW
    NCH = W // _WCH
    NQ = NTW * NCH
    NG = C // _L
    CD = C * D
    assert NT % _NW == 0 and W % _WCH == 0 and C % _L == 0 and D == 16

    def body(x_hbm, tk_hbm, mn_hbm, buf, stg_tk, stg_mn, sem):
        nc = plsc.get_sparse_core_info().num_cores
        wid = lax.axis_index("s") * nc + lax.axis_index("c")

        def chunk_src(q):
            tid = wid + 32 * (q // NCH)
            c = q % NCH
            b = tid // H
            h = tid % H
            ws = pl.multiple_of(c * _WCH, _WCH)
            return x_hbm.at[b, :, h, pl.ds(ws, _WCH), :], b, h, c

        src0, _, _, _ = chunk_src(0)
        pltpu.make_async_copy(src0, buf.at[0], sem).start()

        def q_body(q, carry):
            restk, resmn = carry
            par = lax.rem(q, 2)
            src, b, h, c = chunk_src(q)
            pltpu.make_async_copy(src, buf.at[par], sem).wait()

            @pl.when(q < NQ - 1)
            def _():
                srcn, _, _, _ = chunk_src(q + 1)
                pltpu.make_async_copy(srcn, buf.at[1 - par], sem).start()

            def px_body(pp, cr2):
                restk, resmn = cr2
                groups = []
                tot = jnp.zeros((_L,), jnp.float32)
                for g in range(NG):
                    v = [buf[par, d, pp, pl.ds(g * _L, _L)]
                         for d in range(D)]
                    s = v[0]
                    for d in range(1, D):
                        s = s + v[d]
                    tot = tot + s
                    groups.append(_merge8(_sort8(v[0:8]), _sort8(v[8:16])))
                m01 = _merge8(groups[0], groups[1])
                m23 = _merge8(groups[2], groups[3])
                mall = _merge8(m01, m23)
                ss = [jnp.sort(mall[j]) for j in range(8)]
                r = _xmerge(_xmerge(_xmerge(ss[0], ss[1]),
                                    _xmerge(ss[2], ss[3])),
                            _xmerge(_xmerge(ss[4], ss[5]),
                                    _xmerge(ss[6], ss[7])))
                iota = lax.broadcasted_iota(jnp.int32, (_L,), 0)

                def bcast_sum(vv):
                    cs = plsc.cumsum(vv)
                    head = jnp.where(iota == 0, jnp.flip(cs), 0.0)
                    return plsc.cumsum(head)

                tk_b = bcast_sum(jnp.where(iota >= 8, r, 0.0))
                mn_b = bcast_sum(tot) * (1.0 / CD)
                slot = lax.rem(_WCH * c + pp, _L)
                restk = jnp.where(iota == slot, tk_b, restk)
                resmn = jnp.where(iota == slot, mn_b, resmn)
                return (restk, resmn)

            restk, resmn = lax.fori_loop(0, _WCH, px_body, (restk, resmn))

            @pl.when((lax.rem(c, 2) == 1) | (c == NCH - 1))
            def _():
                off = (c // 2) * _L
                stg_tk[0, pl.ds(off, _L)] = restk
                stg_mn[0, pl.ds(off, _L)] = resmn

            @pl.when(c == NCH - 1)
            def _():
                pltpu.sync_copy(stg_tk, tk_hbm.at[b, h])
                pltpu.sync_copy(stg_mn, mn_hbm.at[b, h])

            return (restk, resmn)

        z = jnp.zeros((_L,), jnp.float32)
        lax.fori_loop(0, NQ, q_body, (z, z))

    mesh = plsc.VectorSubcoreMesh(core_axis_name="c", subcore_axis_name="s")
    return pl.kernel(
        body,
        out_type=[jax.ShapeDtypeStruct((B, H, 8, 64), jnp.float32),
                  jax.ShapeDtypeStruct((B, H, 8, 64), jnp.float32)],
        mesh=mesh,
        compiler_params=pltpu.CompilerParams(needs_layout_passes=False),
        scratch_types=[pltpu.VMEM((2, D, _WCH, C), jnp.float32),
                       pltpu.VMEM((8, 64), jnp.float32),
                       pltpu.VMEM((8, 64), jnp.float32),
                       pltpu.SemaphoreType.DMA],
    )


def kernel(x):
    b, c, d, h, w = x.shape
    xt = jnp.transpose(x, (0, 2, 3, 4, 1))
    tk, mn = _make_sc_call(b, c, d, h, w)(xt)
    tk = tk[:, :, 0, :w].reshape(b, 1, 1, h, w)
    mn = mn[:, :, 0, :w].reshape(b, 1, 1, h, w)
    return (tk, mn)


# speedup vs baseline: 2.7689x; 1.0153x over previous
import jax
import jax.numpy as jnp
from jax import lax
from jax.experimental import pallas as pl
from jax.experimental.pallas import tpu as pltpu
from jax.experimental.pallas import tpu_sc as plsc

_L = 16
_NW = 32
_WCH = 8

_SORT8 = [(0, 1), (2, 3), (4, 5), (6, 7),
          (0, 2), (1, 3), (4, 6), (5, 7),
          (1, 2), (5, 6), (0, 4), (3, 7),
          (1, 5), (2, 6),
          (1, 4), (3, 6),
          (2, 4), (3, 5),
          (3, 4)]

_BITONIC8 = [(0, 4), (1, 5), (2, 6), (3, 7),
             (0, 2), (1, 3), (4, 6), (5, 7),
             (0, 1), (2, 3), (4, 5), (6, 7)]


def _sort8(v):
    v = list(v)
    for a, b in _SORT8:
        hi = jnp.maximum(v[a], v[b])
        lo = jnp.minimum(v[a], v[b])
        v[a] = hi
        v[b] = lo
    return v


def _merge8(a, b):
    m = [jnp.maximum(a[j], b[7 - j]) for j in range(8)]
    for p, q in _BITONIC8:
        hi = jnp.maximum(m[p], m[q])
        lo = jnp.minimum(m[p], m[q])
        m[p] = hi
        m[q] = lo
    return m


def _xmerge(a, b):
    return jnp.sort(jnp.maximum(a, jnp.flip(b)))


def _make_sc_call(B, C, D, H, W):
    NT = B * H
    NTW = NT // _NW
    NCH = W // _WCH
    NQ = NTW * NCH
    NG = C // _L
    CD = C * D
    assert NT % _NW == 0 and W % _WCH == 0 and C % _L == 0 and D == 16

    def body(x_hbm, tk_hbm, mn_hbm, buf, stg_tk, stg_mn, sem):
        nc = plsc.get_sparse_core_info().num_cores
        wid = lax.axis_index("s") * nc + lax.axis_index("c")

        def chunk_src(q):
            tid = wid + 32 * (q // NCH)
            c = q % NCH
            b = tid // H
            h = tid % H
            ws = pl.multiple_of(c * _WCH, _WCH)
            return x_hbm.at[b, :, h, pl.ds(ws, _WCH), :], b, h, c

        src0, _, _, _ = chunk_src(0)
        pltpu.make_async_copy(src0, buf.at[0], sem).start()

        def q_body(q, carry):
            restk, resmn = carry
            par = lax.rem(q, 2)
            src, b, h, c = chunk_src(q)
            pltpu.make_async_copy(src, buf.at[par], sem).wait()

            @pl.when(q < NQ - 1)
            def _():
                srcn, _, _, _ = chunk_src(q + 1)
                pltpu.make_async_copy(srcn, buf.at[1 - par], sem).start()

            iota = lax.broadcasted_iota(jnp.int32, (_L,), 0)

            def bcast_sum(vv):
                cs = plsc.cumsum(vv)
                head = jnp.where(iota == 0, jnp.flip(cs), 0.0)
                return plsc.cumsum(head)

            def one_px(pp):
                groups = []
                tot = jnp.zeros((_L,), jnp.float32)
                for g in range(NG):
                    v = [buf[par, d, pp, pl.ds(g * _L, _L)]
                         for d in range(D)]
                    s = v[0]
                    for d in range(1, D):
                        s = s + v[d]
                    tot = tot + s
                    groups.append(_merge8(_sort8(v[0:8]), _sort8(v[8:16])))
                m01 = _merge8(groups[0], groups[1])
                m23 = _merge8(groups[2], groups[3])
                mall = _merge8(m01, m23)
                ss = [jnp.sort(mall[j]) for j in range(8)]
                r = _xmerge(_xmerge(_xmerge(ss[0], ss[1]),
                                    _xmerge(ss[2], ss[3])),
                            _xmerge(_xmerge(ss[4], ss[5]),
                                    _xmerge(ss[6], ss[7])))
                tk_b = bcast_sum(jnp.where(iota >= 8, r, 0.0))
                mn_b = bcast_sum(tot) * (1.0 / CD)
                return tk_b, mn_b

            def px_body(pe, cr2):
                restk, resmn = cr2
                for u in range(2):
                    pp = 2 * pe + u
                    tk_b, mn_b = one_px(pp)
                    slot = lax.rem(_WCH * c + pp, _L)
                    restk = jnp.where(iota == slot, tk_b, restk)
                    resmn = jnp.where(iota == slot, mn_b, resmn)
                return (restk, resmn)

            restk, resmn = lax.fori_loop(0, _WCH // 2, px_body,
                                         (restk, resmn))

            @pl.when((lax.rem(c, 2) == 1) | (c == NCH - 1))
            def _():
                off = (c // 2) * _L
                stg_tk[0, pl.ds(off, _L)] = restk
                stg_mn[0, pl.ds(off, _L)] = resmn

            @pl.when(c == NCH - 1)
            def _():
                pltpu.sync_copy(stg_tk, tk_hbm.at[b, h])
                pltpu.sync_copy(stg_mn, mn_hbm.at[b, h])

            return (restk, resmn)

        z = jnp.zeros((_L,), jnp.float32)
        lax.fori_loop(0, NQ, q_body, (z, z))

    mesh = plsc.VectorSubcoreMesh(core_axis_name="c", subcore_axis_name="s")
    return pl.kernel(
        body,
        out_type=[jax.ShapeDtypeStruct((B, H, 8, 64), jnp.float32),
                  jax.ShapeDtypeStruct((B, H, 8, 64), jnp.float32)],
        mesh=mesh,
        compiler_params=pltpu.CompilerParams(needs_layout_passes=False),
        scratch_types=[pltpu.VMEM((2, D, _WCH, C), jnp.float32),
                       pltpu.VMEM((8, 64), jnp.float32),
                       pltpu.VMEM((8, 64), jnp.float32),
                       pltpu.SemaphoreType.DMA],
    )


def kernel(x):
    b, c, d, h, w = x.shape
    xt = jnp.transpose(x, (0, 2, 3, 4, 1))
    tk, mn = _make_sc_call(b, c, d, h, w)(xt)
    tk = tk[:, :, 0, :w].reshape(b, 1, 1, h, w)
    mn = mn[:, :, 0, :w].reshape(b, 1, 1, h, w)
    return (tk, mn)


# speedup vs baseline: 3.1043x; 1.1211x over previous
import jax
import jax.numpy as jnp
from jax import lax
from jax.experimental import pallas as pl
from jax.experimental.pallas import tpu as pltpu
from jax.experimental.pallas import tpu_sc as plsc

_L = 16
_NW = 32
_WCH = 8

_SORT8 = [(0, 1), (2, 3), (4, 5), (6, 7),
          (0, 2), (1, 3), (4, 6), (5, 7),
          (1, 2), (5, 6), (0, 4), (3, 7),
          (1, 5), (2, 6),
          (1, 4), (3, 6),
          (2, 4), (3, 5),
          (3, 4)]

_BITONIC8 = [(0, 4), (1, 5), (2, 6), (3, 7),
             (0, 2), (1, 3), (4, 6), (5, 7),
             (0, 1), (2, 3), (4, 5), (6, 7)]


def _sort8(v):
    v = list(v)
    for a, b in _SORT8:
        hi = jnp.maximum(v[a], v[b])
        lo = jnp.minimum(v[a], v[b])
        v[a] = hi
        v[b] = lo
    return v


def _merge8(a, b):
    m = [jnp.maximum(a[j], b[7 - j]) for j in range(8)]
    for p, q in _BITONIC8:
        hi = jnp.maximum(m[p], m[q])
        lo = jnp.minimum(m[p], m[q])
        m[p] = hi
        m[q] = lo
    return m


def _merge8_top(a, b):
    return [jnp.maximum(a[j], b[7 - j]) for j in range(8)]


def _xmerge(a, b):
    return jnp.sort(jnp.maximum(a, jnp.flip(b)))


def _make_sc_call(B, C, D, H, W):
    NT = B * H
    NTW = NT // _NW
    NCH = W // _WCH
    NQ = NTW * NCH
    NG = C // _L
    CD = C * D
    assert NT % _NW == 0 and W % _WCH == 0 and C % _L == 0 and D == 16

    def body(x_hbm, tk_hbm, mn_hbm, buf, stg_tk, stg_mn, sem):
        nc = plsc.get_sparse_core_info().num_cores
        wid = lax.axis_index("s") * nc + lax.axis_index("c")

        def chunk_src(q):
            tid = wid + 32 * (q // NCH)
            c = q % NCH
            b = tid // H
            h = tid % H
            ws = pl.multiple_of(c * _WCH, _WCH)
            return x_hbm.at[b, :, h, pl.ds(ws, _WCH), :], b, h, c

        src0, _, _, _ = chunk_src(0)
        pltpu.make_async_copy(src0, buf.at[0], sem).start()

        def q_body(q, carry):
            restk, resmn = carry
            par = lax.rem(q, 2)
            src, b, h, c = chunk_src(q)
            pltpu.make_async_copy(src, buf.at[par], sem).wait()

            @pl.when(q < NQ - 1)
            def _():
                srcn, _, _, _ = chunk_src(q + 1)
                pltpu.make_async_copy(srcn, buf.at[1 - par], sem).start()

            iota = lax.broadcasted_iota(jnp.int32, (_L,), 0)

            def bcast_sum(vv):
                cs = plsc.cumsum(vv)
                head = jnp.where(iota == 0, jnp.flip(cs), 0.0)
                return plsc.cumsum(head)

            def one_px(pp):
                groups = []
                tot = jnp.zeros((_L,), jnp.float32)
                for g in range(NG):
                    v = [buf[par, d, pp, pl.ds(g * _L, _L)]
                         for d in range(D)]
                    s01 = (v[0] + v[1]) + (v[2] + v[3])
                    s23 = (v[4] + v[5]) + (v[6] + v[7])
                    s45 = (v[8] + v[9]) + (v[10] + v[11])
                    s67 = (v[12] + v[13]) + (v[14] + v[15])
                    tot = tot + ((s01 + s23) + (s45 + s67))
                    groups.append(_merge8(_sort8(v[0:8]), _sort8(v[8:16])))
                m01 = _merge8(groups[0], groups[1])
                m23 = _merge8(groups[2], groups[3])
                mall = _merge8_top(m01, m23)
                ss = [jnp.sort(mall[j]) for j in range(8)]
                r = _xmerge(_xmerge(_xmerge(ss[0], ss[1]),
                                    _xmerge(ss[2], ss[3])),
                            _xmerge(_xmerge(ss[4], ss[5]),
                                    _xmerge(ss[6], ss[7])))
                tk_b = bcast_sum(jnp.where(iota >= 8, r, 0.0))
                mn_b = bcast_sum(tot) * (1.0 / CD)
                return tk_b, mn_b

            def px_body(pe, cr2):
                restk, resmn = cr2
                for u in range(2):
                    pp = 2 * pe + u
                    tk_b, mn_b = one_px(pp)
                    slot = lax.rem(_WCH * c + pp, _L)
                    restk = jnp.where(iota == slot, tk_b, restk)
                    resmn = jnp.where(iota == slot, mn_b, resmn)
                return (restk, resmn)

            restk, resmn = lax.fori_loop(0, _WCH // 2, px_body,
                                         (restk, resmn))

            @pl.when((lax.rem(c, 2) == 1) | (c == NCH - 1))
            def _():
                off = (c // 2) * _L
                stg_tk[0, pl.ds(off, _L)] = restk
                stg_mn[0, pl.ds(off, _L)] = resmn

            @pl.when(c == NCH - 1)
            def _():
                pltpu.sync_copy(stg_tk, tk_hbm.at[b, h])
                pltpu.sync_copy(stg_mn, mn_hbm.at[b, h])

            return (restk, resmn)

        z = jnp.zeros((_L,), jnp.float32)
        lax.fori_loop(0, NQ, q_body, (z, z))

    mesh = plsc.VectorSubcoreMesh(core_axis_name="c", subcore_axis_name="s")
    return pl.kernel(
        body,
        out_type=[jax.ShapeDtypeStruct((B, H, 8, 64), jnp.float32),
                  jax.ShapeDtypeStruct((B, H, 8, 64), jnp.float32)],
        mesh=mesh,
        compiler_params=pltpu.CompilerParams(needs_layout_passes=False),
        scratch_types=[pltpu.VMEM((2, D, _WCH, C), jnp.float32),
                       pltpu.VMEM((8, 64), jnp.float32),
                       pltpu.VMEM((8, 64), jnp.float32),
                       pltpu.SemaphoreType.DMA],
    )


def kernel(x):
    b, c, d, h, w = x.shape
    xt = jnp.transpose(x, (0, 2, 3, 4, 1))
    tk, mn = _make_sc_call(b, c, d, h, w)(xt)
    tk = tk[:, :, 0, :w].reshape(b, 1, 1, h, w)
    mn = mn[:, :, 0, :w].reshape(b, 1, 1, h, w)
    return (tk, mn)
